# baseline (speedup 1.0000x reference)
import functools
import math

import jax
import jax.numpy as jnp
from jax import lax
from jax.experimental import pallas as pl
from jax.experimental.pallas import tpu as pltpu
from jax.experimental.pallas import tpu_sc as plsc

NLAYERS = 4
NFIX = 2
N = 10000
E = 320000
C = 128
L = 16
NC = 2
NS = 16
NW = NC * NS
EPW = E // NW
CH = 40
NCH = EPW // CH
NP = 10112
RPS = NP // NS


def _sc_edge_body(dz_hbm, ndz_hbm, idx3_hbm, ewn_hbm, zrows_hbm, out_hbm,
                  aggsh, *bufs):
    idxbs = list(bufs[0:12])
    ewcs = list(bufs[12:18])
    avs = list(bufs[18:24])
    sems = list(bufs[24:54])
    sis = sems[0::5]
    ses = sems[1::5]
    sas = sems[2::5]
    sbs = sems[3::5]
    sss = sems[4::5]

    cid = lax.axis_index("c")
    sid = lax.axis_index("s")
    wid = cid * NS + sid
    ebase = wid * EPW
    cgbase = wid * NCH
    rbase = sid * RPS

    def fire_idx(c, s6, s12, i):
        pltpu.async_copy(idx3_hbm.at[i, cgbase + c], idxbs[s12], sis[s6])
        pltpu.async_copy(ewn_hbm.at[pl.ds((ebase + c * CH) * L, CH * L)],
                         ewcs[s6], ses[s6])

    def wait_scatter(s6, s12):
        pltpu.make_async_copy(avs[s6], aggsh.at[idxbs[s12].at[0]],
                              sss[s6]).wait()

    def g1_stage(c, s6, s12, i):
        pltpu.make_async_copy(idx3_hbm.at[i, cgbase + c], idxbs[s12],
                              sis[s6]).wait()
        pltpu.async_copy(ndz_hbm.at[idxbs[s12].at[2]], avs[s6], sas[s6])

    def g2_stage(s6, s12):
        pltpu.make_async_copy(ndz_hbm.at[idxbs[s12].at[2]], avs[s6],
                              sas[s6]).wait()
        pltpu.async_copy(dz_hbm.at[idxbs[s12].at[1]], avs[s6], sbs[s6],
                         add=True)

    def compute_stage(c, s6, s12):
        av = avs[s6]
        ewc = ewcs[s6]
        pltpu.make_async_copy(dz_hbm.at[idxbs[s12].at[1]], av,
                              sbs[s6]).wait()
        pltpu.make_async_copy(ewn_hbm.at[pl.ds((ebase + c * CH) * L, CH * L)],
                              ewc, ses[s6]).wait()

        def edge_body(e, _):
            ews = ewc[pl.ds(e * L, L)]
            for k in range(C // L):
                sl = pl.ds(k * L, L)
                mg = av[e, sl] * ews
                den = 1.0 + jnp.exp(mg)
                av[e, sl] = -(mg / den)
            return 0

        lax.fori_loop(0, CH, edge_body, 0)
        pltpu.async_copy(av, aggsh.at[idxbs[s12].at[0]], sss[s6], add=True)

    NBODY = NCH + 5
    NTRIP = (NBODY + 11) // 12

    for i in range(NLAYERS):
        pltpu.sync_copy(zrows_hbm, aggsh.at[pl.ds(rbase, RPS)])
        plsc.subcore_barrier()

        def trip_body(p, _, i=i):
            for bb in range(12):
                n = p * 12 + bb

                @pl.when(jnp.logical_and(n - 5 >= 0, n - 5 < NCH))
                def _(n=n, bb=bb):
                    compute_stage(n - 5, (bb - 5) % 6, (bb - 5) % 12)

                @pl.when(jnp.logical_and(n - 4 >= 0, n - 4 < NCH))
                def _(n=n, bb=bb):
                    g2_stage((bb - 4) % 6, (bb - 4) % 12)

                @pl.when(jnp.logical_and(n - 2 >= 0, n - 2 < NCH))
                def _(n=n, bb=bb, i=i):
                    @pl.when(n - 8 >= 0)
                    def _(bb=bb):
                        wait_scatter((bb - 8) % 6, (bb - 8) % 12)
                    g1_stage(n - 2, (bb - 2) % 6, (bb - 2) % 12, i)

                @pl.when(n < NCH)
                def _(n=n, bb=bb, i=i):
                    fire_idx(n, bb % 6, bb % 12, i)
            return 0

        lax.fori_loop(0, NTRIP, trip_body, 0)
        for c in range(NCH - 6, NCH):
            wait_scatter(c % 6, c % 12)
        plsc.subcore_barrier()
        pltpu.sync_copy(aggsh.at[pl.ds(rbase, RPS)],
                        out_hbm.at[cid, i, pl.ds(rbase, RPS)])


_sc_edge = functools.partial(
    pl.kernel,
    out_type=jax.ShapeDtypeStruct((NC, NLAYERS, NP, C), jnp.float32),
    mesh=plsc.VectorSubcoreMesh(core_axis_name="c", subcore_axis_name="s",
                                num_cores=NC, num_subcores=NS),
    scratch_types=(
        [pltpu.VMEM_SHARED((NP, C), jnp.float32)]
        + [pltpu.VMEM((3, CH), jnp.int32)] * 12
        + [pltpu.VMEM((CH * L,), jnp.float32)] * 6
        + [pltpu.VMEM((CH, C), jnp.float32)] * 6
        + [pltpu.SemaphoreType.DMA] * 30
    ),
)(_sc_edge_body)


@jax.jit
def _edge_op(dZ, nDZ, idx3, ewn, zrows):
    return _sc_edge(dZ, nDZ, idx3, ewn, zrows)


def _silu(x):
    return x * jax.nn.sigmoid(x)


BN = 1000


def _dense_in_body(z_ref, f_ref, kf_ref, k_ref, dz_ref, ndz_ref):
    z = z_ref[0]
    fb = f_ref[...]
    kf = kf_ref[0]
    zc = (jnp.dot(z, kf[:C], preferred_element_type=jnp.float32)
          + jnp.dot(fb, kf[C:], preferred_element_type=jnp.float32))
    zc = _silu(zc)
    dz = jnp.dot(zc, k_ref[0], preferred_element_type=jnp.float32)
    dz_ref[0] = dz
    ndz_ref[0] = -dz


@jax.jit
def _dense_in(Z, f, Kf, K):
    return pl.pallas_call(
        _dense_in_body,
        grid=(NLAYERS, N // BN),
        in_specs=[
            pl.BlockSpec((1, BN, C), lambda i, j: (i, j, 0)),
            pl.BlockSpec((BN, C), lambda i, j: (j, 0)),
            pl.BlockSpec((1, 2 * C, C), lambda i, j: (i, 0, 0)),
            pl.BlockSpec((1, C, C), lambda i, j: (i, 0, 0)),
        ],
        out_specs=[
            pl.BlockSpec((1, BN, C), lambda i, j: (i, j, 0)),
            pl.BlockSpec((1, BN, C), lambda i, j: (i, j, 0)),
        ],
        out_shape=[
            jax.ShapeDtypeStruct((NLAYERS, N, C), jnp.float32),
            jax.ShapeDtypeStruct((NLAYERS, N, C), jnp.float32),
        ],
    )(Z, f, Kf, K)


def _dense_out_body(pa_ref, pb_ref, k_ref, x_ref, z_ref):
    Ys = []
    for i in range(NLAYERS):
        agg = pa_ref[0, i] + pb_ref[0, i]
        y = -lax.dot_general(agg, k_ref[i], (((1,), (1,)), ((), ())),
                             preferred_element_type=jnp.float32)
        if i == NLAYERS - 1:
            y = y + x_ref[...]
        Ys.append(y)
    Yt = [None] * NLAYERS
    Yt[0] = math.sqrt(0.5) * Ys[0]
    for i in range(1, NLAYERS):
        a = math.sqrt((i + 1) / (i + 2))
        b = math.sqrt(i / (i + 1))
        Yt[i] = a * (b * Yt[i - 1] + Ys[i])
    W = [None] * NLAYERS
    W[NLAYERS - 1] = math.sqrt(NLAYERS / (NLAYERS + 1)) * Yt[NLAYERS - 1]
    for i in range(NLAYERS - 2, -1, -1):
        a = math.sqrt((i + 1) / (i + 2))
        W[i] = a * (a * W[i + 1] + Yt[i])
    for i in range(NLAYERS):
        z_ref[i] = W[i]


@jax.jit
def _dense_out(parts, K, X):
    return pl.pallas_call(
        _dense_out_body,
        grid=(N // BN,),
        in_specs=[
            pl.BlockSpec((1, NLAYERS, BN, C), lambda j: (0, 0, j, 0)),
            pl.BlockSpec((1, NLAYERS, BN, C), lambda j: (1, 0, j, 0)),
            pl.BlockSpec((NLAYERS, C, C), lambda j: (0, 0, 0)),
            pl.BlockSpec((BN, C), lambda j: (j, 0)),
        ],
        out_specs=pl.BlockSpec((NLAYERS, BN, C), lambda j: (0, j, 0)),
        out_shape=jax.ShapeDtypeStruct((NLAYERS, N, C), jnp.float32),
    )(parts, parts, K, X)


def kernel(X, f, edge_weight, K, K_features, edge_index):
    src = edge_index[0]
    dst = edge_index[1]
    zrows = jnp.zeros((RPS, C), jnp.float32)
    ewn = jnp.repeat(-edge_weight[:, None], L, axis=1).reshape(E * L)
    srcr = src.reshape(E // CH, CH)
    dstr = dst.reshape(E // CH, CH)
    idx3 = jnp.stack([
        jnp.stack([srcr, srcr + i * N, dstr + i * N], axis=1)
        for i in range(NLAYERS)
    ])
    Z = jnp.zeros((NLAYERS, N, C), dtype=jnp.float32)
    for _ in range(NFIX):
        dZ, nDZ = _dense_in(Z, f, K_features, K)
        parts = _edge_op(dZ.reshape(NLAYERS * N, C),
                         nDZ.reshape(NLAYERS * N, C), idx3, ewn, zrows)
        Z = _dense_out(parts, K, X)
    return (Z[-1], Z)


# speedup vs baseline: 1.1223x; 1.1223x over previous
import functools
import math

import jax
import jax.numpy as jnp
from jax import lax
from jax.experimental import pallas as pl
from jax.experimental.pallas import tpu as pltpu
from jax.experimental.pallas import tpu_sc as plsc

NLAYERS = 4
NFIX = 2
N = 10000
E = 320000
C = 128
L = 16
NC = 2
NS = 16
NW = NC * NS
EPW = E // NW
CH = 80
NCH = EPW // CH
NP = 10112
RPS = NP // NS


def _sc_edge_body(dz_hbm, ndz_hbm, idx3_hbm, ewn_hbm, zrows_hbm, out_hbm,
                  aggsh, *bufs):
    idxbs = list(bufs[0:6])
    ewcs = list(bufs[6:9])
    avs = list(bufs[9:12])
    sems = list(bufs[12:27])
    sis = sems[0::5]
    ses = sems[1::5]
    sas = sems[2::5]
    sbs = sems[3::5]
    sss = sems[4::5]

    cid = lax.axis_index("c")
    sid = lax.axis_index("s")
    wid = cid * NS + sid
    ebase = wid * EPW
    cgbase = wid * NCH
    rbase = sid * RPS

    def fire_idx(c, s6, s12, i):
        pltpu.async_copy(idx3_hbm.at[i, cgbase + c], idxbs[s12], sis[s6])
        pltpu.async_copy(ewn_hbm.at[pl.ds((ebase + c * CH) * L, CH * L)],
                         ewcs[s6], ses[s6])

    def wait_scatter(s6, s12):
        pltpu.make_async_copy(avs[s6], aggsh.at[idxbs[s12].at[0]],
                              sss[s6]).wait()

    def g1_stage(c, s6, s12, i):
        pltpu.make_async_copy(idx3_hbm.at[i, cgbase + c], idxbs[s12],
                              sis[s6]).wait()
        pltpu.async_copy(ndz_hbm.at[idxbs[s12].at[2]], avs[s6], sas[s6])

    def g2_stage(s6, s12):
        pltpu.make_async_copy(ndz_hbm.at[idxbs[s12].at[2]], avs[s6],
                              sas[s6]).wait()
        pltpu.async_copy(dz_hbm.at[idxbs[s12].at[1]], avs[s6], sbs[s6],
                         add=True)

    def compute_stage(c, s6, s12):
        av = avs[s6]
        ewc = ewcs[s6]
        pltpu.make_async_copy(dz_hbm.at[idxbs[s12].at[1]], av,
                              sbs[s6]).wait()
        pltpu.make_async_copy(ewn_hbm.at[pl.ds((ebase + c * CH) * L, CH * L)],
                              ewc, ses[s6]).wait()

        def edge_body(e, _):
            ews = ewc[pl.ds(e * L, L)]
            for k in range(C // L):
                sl = pl.ds(k * L, L)
                mg = av[e, sl] * ews
                den = 1.0 + jnp.exp(mg)
                av[e, sl] = -(mg / den)
            return 0

        lax.fori_loop(0, CH, edge_body, 0)
        pltpu.async_copy(av, aggsh.at[idxbs[s12].at[0]], sss[s6], add=True)

    NBODY = NCH + 3
    NTRIP = (NBODY + 5) // 6

    for i in range(NLAYERS):
        pltpu.sync_copy(zrows_hbm, aggsh.at[pl.ds(rbase, RPS)])
        plsc.subcore_barrier()

        def trip_body(p, _, i=i):
            for bb in range(6):
                n = p * 6 + bb

                @pl.when(jnp.logical_and(n - 3 >= 0, n - 3 < NCH))
                def _(n=n, bb=bb):
                    compute_stage(n - 3, (bb - 3) % 3, (bb - 3) % 6)

                @pl.when(jnp.logical_and(n - 2 >= 0, n - 2 < NCH))
                def _(n=n, bb=bb):
                    g2_stage((bb - 2) % 3, (bb - 2) % 6)

                @pl.when(jnp.logical_and(n - 1 >= 0, n - 1 < NCH))
                def _(n=n, bb=bb, i=i):
                    @pl.when(n - 4 >= 0)
                    def _(bb=bb):
                        wait_scatter((bb - 4) % 3, (bb - 4) % 6)
                    g1_stage(n - 1, (bb - 1) % 3, (bb - 1) % 6, i)

                @pl.when(n < NCH)
                def _(n=n, bb=bb, i=i):
                    fire_idx(n, bb % 3, bb % 6, i)
            return 0

        lax.fori_loop(0, NTRIP, trip_body, 0)
        for c in range(NCH - 3, NCH):
            wait_scatter(c % 3, c % 6)
        plsc.subcore_barrier()
        pltpu.sync_copy(aggsh.at[pl.ds(rbase, RPS)],
                        out_hbm.at[cid, i, pl.ds(rbase, RPS)])


_sc_edge = functools.partial(
    pl.kernel,
    out_type=jax.ShapeDtypeStruct((NC, NLAYERS, NP, C), jnp.float32),
    mesh=plsc.VectorSubcoreMesh(core_axis_name="c", subcore_axis_name="s",
                                num_cores=NC, num_subcores=NS),
    scratch_types=(
        [pltpu.VMEM_SHARED((NP, C), jnp.float32)]
        + [pltpu.VMEM((3, CH), jnp.int32)] * 6
        + [pltpu.VMEM((CH * L,), jnp.float32)] * 3
        + [pltpu.VMEM((CH, C), jnp.float32)] * 3
        + [pltpu.SemaphoreType.DMA] * 15
    ),
)(_sc_edge_body)


@jax.jit
def _edge_op(dZ, nDZ, idx3, ewn, zrows):
    return _sc_edge(dZ, nDZ, idx3, ewn, zrows)


def _silu(x):
    return x * jax.nn.sigmoid(x)


BN = 1000


def _dense_in_body(z_ref, f_ref, kf_ref, k_ref, dz_ref, ndz_ref):
    z = z_ref[0]
    fb = f_ref[...]
    kf = kf_ref[0]
    zc = (jnp.dot(z, kf[:C], preferred_element_type=jnp.float32)
          + jnp.dot(fb, kf[C:], preferred_element_type=jnp.float32))
    zc = _silu(zc)
    dz = jnp.dot(zc, k_ref[0], preferred_element_type=jnp.float32)
    dz_ref[0] = dz
    ndz_ref[0] = -dz


@jax.jit
def _dense_in(Z, f, Kf, K):
    return pl.pallas_call(
        _dense_in_body,
        grid=(NLAYERS, N // BN),
        in_specs=[
            pl.BlockSpec((1, BN, C), lambda i, j: (i, j, 0)),
            pl.BlockSpec((BN, C), lambda i, j: (j, 0)),
            pl.BlockSpec((1, 2 * C, C), lambda i, j: (i, 0, 0)),
            pl.BlockSpec((1, C, C), lambda i, j: (i, 0, 0)),
        ],
        out_specs=[
            pl.BlockSpec((1, BN, C), lambda i, j: (i, j, 0)),
            pl.BlockSpec((1, BN, C), lambda i, j: (i, j, 0)),
        ],
        out_shape=[
            jax.ShapeDtypeStruct((NLAYERS, N, C), jnp.float32),
            jax.ShapeDtypeStruct((NLAYERS, N, C), jnp.float32),
        ],
    )(Z, f, Kf, K)


def _dense_out_body(pa_ref, pb_ref, k_ref, x_ref, z_ref):
    Ys = []
    for i in range(NLAYERS):
        agg = pa_ref[0, i] + pb_ref[0, i]
        y = -lax.dot_general(agg, k_ref[i], (((1,), (1,)), ((), ())),
                             preferred_element_type=jnp.float32)
        if i == NLAYERS - 1:
            y = y + x_ref[...]
        Ys.append(y)
    Yt = [None] * NLAYERS
    Yt[0] = math.sqrt(0.5) * Ys[0]
    for i in range(1, NLAYERS):
        a = math.sqrt((i + 1) / (i + 2))
        b = math.sqrt(i / (i + 1))
        Yt[i] = a * (b * Yt[i - 1] + Ys[i])
    W = [None] * NLAYERS
    W[NLAYERS - 1] = math.sqrt(NLAYERS / (NLAYERS + 1)) * Yt[NLAYERS - 1]
    for i in range(NLAYERS - 2, -1, -1):
        a = math.sqrt((i + 1) / (i + 2))
        W[i] = a * (a * W[i + 1] + Yt[i])
    for i in range(NLAYERS):
        z_ref[i] = W[i]


@jax.jit
def _dense_out(parts, K, X):
    return pl.pallas_call(
        _dense_out_body,
        grid=(N // BN,),
        in_specs=[
            pl.BlockSpec((1, NLAYERS, BN, C), lambda j: (0, 0, j, 0)),
            pl.BlockSpec((1, NLAYERS, BN, C), lambda j: (1, 0, j, 0)),
            pl.BlockSpec((NLAYERS, C, C), lambda j: (0, 0, 0)),
            pl.BlockSpec((BN, C), lambda j: (j, 0)),
        ],
        out_specs=pl.BlockSpec((NLAYERS, BN, C), lambda j: (0, j, 0)),
        out_shape=jax.ShapeDtypeStruct((NLAYERS, N, C), jnp.float32),
    )(parts, parts, K, X)


def kernel(X, f, edge_weight, K, K_features, edge_index):
    src = edge_index[0]
    dst = edge_index[1]
    zrows = jnp.zeros((RPS, C), jnp.float32)
    ewn = jnp.repeat(-edge_weight[:, None], L, axis=1).reshape(E * L)
    srcr = src.reshape(E // CH, CH)
    dstr = dst.reshape(E // CH, CH)
    idx3 = jnp.stack([
        jnp.stack([srcr, srcr + i * N, dstr + i * N], axis=1)
        for i in range(NLAYERS)
    ])
    Z = jnp.zeros((NLAYERS, N, C), dtype=jnp.float32)
    for _ in range(NFIX):
        dZ, nDZ = _dense_in(Z, f, K_features, K)
        parts = _edge_op(dZ.reshape(NLAYERS * N, C),
                         nDZ.reshape(NLAYERS * N, C), idx3, ewn, zrows)
        Z = _dense_out(parts, K, X)
    return (Z[-1], Z)


# speedup vs baseline: 1.1225x; 1.0002x over previous
import functools
import math

import jax
import jax.numpy as jnp
from jax import lax
from jax.experimental import pallas as pl
from jax.experimental.pallas import tpu as pltpu
from jax.experimental.pallas import tpu_sc as plsc

NLAYERS = 4
NFIX = 2
N = 10000
E = 320000
C = 128
L = 16
NC = 2
NS = 16
NW = NC * NS
EPW = E // NW
CH = 80
NCH = EPW // CH
NP = 10112
RPS = NP // NS


def _sc_edge_body(dz_hbm, ndz_hbm, idx3_hbm, ewn_hbm, zrows_hbm, out_hbm,
                  aggsh, *bufs):
    idxbs = list(bufs[0:6])
    ewcs = list(bufs[6:9])
    avs = list(bufs[9:12])
    sems = list(bufs[12:27])
    sis = sems[0::5]
    ses = sems[1::5]
    sas = sems[2::5]
    sbs = sems[3::5]
    sss = sems[4::5]

    cid = lax.axis_index("c")
    sid = lax.axis_index("s")
    wid = cid * NS + sid
    ebase = wid * EPW
    cgbase = wid * NCH
    rbase = sid * RPS

    def fire_idx(c, r3, r6, i):
        pltpu.async_copy(idx3_hbm.at[i, cgbase + c], idxbs[r6], sis[r3])
        pltpu.async_copy(ewn_hbm.at[pl.ds((ebase + c * CH) * L, CH * L)],
                         ewcs[r3], ses[r3])

    def wait_scatter(r3, r6):
        pltpu.make_async_copy(avs[r3], aggsh.at[idxbs[r6].at[0]],
                              sss[r3]).wait()

    def g1_stage(c, r3, r6, i):
        pltpu.make_async_copy(idx3_hbm.at[i, cgbase + c], idxbs[r6],
                              sis[r3]).wait()
        pltpu.async_copy(ndz_hbm.at[idxbs[r6].at[2]], avs[r3], sas[r3])

    def g2_stage(r3, r6):
        pltpu.make_async_copy(ndz_hbm.at[idxbs[r6].at[2]], avs[r3],
                              sas[r3]).wait()
        pltpu.async_copy(dz_hbm.at[idxbs[r6].at[1]], avs[r3], sbs[r3],
                         add=True)

    def compute_stage(c, r3, r6):
        av = avs[r3]
        ewc = ewcs[r3]
        pltpu.make_async_copy(dz_hbm.at[idxbs[r6].at[1]], av,
                              sbs[r3]).wait()
        pltpu.make_async_copy(ewn_hbm.at[pl.ds((ebase + c * CH) * L, CH * L)],
                              ewc, ses[r3]).wait()

        def edge_body(e, _):
            ews = ewc[pl.ds(e * L, L)]
            for k in range(C // L):
                sl = pl.ds(k * L, L)
                mg = av[e, sl] * ews
                den = 1.0 + jnp.exp(mg)
                av[e, sl] = -(mg / den)
            return 0

        lax.fori_loop(0, CH, edge_body, 0)
        pltpu.async_copy(av, aggsh.at[idxbs[r6].at[0]], sss[r3], add=True)

    NBODY = NCH + 3
    NTRIP = (NBODY + 5) // 6

    for i in range(NLAYERS):
        pltpu.sync_copy(zrows_hbm, aggsh.at[pl.ds(rbase, RPS)])
        plsc.subcore_barrier()

        def trip_body(p, _, i=i):
            for bb in range(6):
                n = p * 6 + bb

                @pl.when(jnp.logical_and(n - 3 >= 0, n - 3 < NCH))
                def _(n=n, bb=bb):
                    compute_stage(n - 3, (bb - 3) % 3, (bb - 3) % 6)

                @pl.when(jnp.logical_and(n - 2 >= 0, n - 2 < NCH))
                def _(n=n, bb=bb):
                    g2_stage((bb - 2) % 3, (bb - 2) % 6)

                @pl.when(jnp.logical_and(n - 1 >= 0, n - 1 < NCH))
                def _(n=n, bb=bb, i=i):
                    @pl.when(n - 4 >= 0)
                    def _(bb=bb):
                        wait_scatter((bb - 4) % 3, (bb - 4) % 6)
                    g1_stage(n - 1, (bb - 1) % 3, (bb - 1) % 6, i)

                @pl.when(n < NCH)
                def _(n=n, bb=bb, i=i):
                    fire_idx(n, bb % 3, bb % 6, i)
            return 0

        lax.fori_loop(0, NTRIP, trip_body, 0)
        for c in range(NCH - 3, NCH):
            wait_scatter(c % 3, c % 6)
        plsc.subcore_barrier()
        pltpu.sync_copy(aggsh.at[pl.ds(rbase, RPS)],
                        out_hbm.at[cid, i, pl.ds(rbase, RPS)])


_sc_edge = functools.partial(
    pl.kernel,
    out_type=jax.ShapeDtypeStruct((NC, NLAYERS, NP, C), jnp.float32),
    mesh=plsc.VectorSubcoreMesh(core_axis_name="c", subcore_axis_name="s",
                                num_cores=NC, num_subcores=NS),
    scratch_types=(
        [pltpu.VMEM_SHARED((NP, C), jnp.float32)]
        + [pltpu.VMEM((3, CH), jnp.int32)] * 6
        + [pltpu.VMEM((CH * L,), jnp.float32)] * 3
        + [pltpu.VMEM((CH, C), jnp.float32)] * 3
        + [pltpu.SemaphoreType.DMA] * 15
    ),
)(_sc_edge_body)


@jax.jit
def _edge_op(dZ, nDZ, idx3, ewn, zrows):
    return _sc_edge(dZ, nDZ, idx3, ewn, zrows)


def _silu(x):
    return x * jax.nn.sigmoid(x)


BN = 1000


def _dense_in_body(z_ref, f_ref, kf_ref, k_ref, dz_ref, ndz_ref):
    z = z_ref[0]
    fb = f_ref[...]
    kf = kf_ref[0]
    zc = (jnp.dot(z, kf[:C], preferred_element_type=jnp.float32)
          + jnp.dot(fb, kf[C:], preferred_element_type=jnp.float32))
    zc = _silu(zc)
    dz = jnp.dot(zc, k_ref[0], preferred_element_type=jnp.float32)
    dz_ref[0] = dz
    ndz_ref[0] = -dz


@jax.jit
def _dense_in(Z, f, Kf, K):
    return pl.pallas_call(
        _dense_in_body,
        grid=(NLAYERS, N // BN),
        in_specs=[
            pl.BlockSpec((1, BN, C), lambda i, j: (i, j, 0)),
            pl.BlockSpec((BN, C), lambda i, j: (j, 0)),
            pl.BlockSpec((1, 2 * C, C), lambda i, j: (i, 0, 0)),
            pl.BlockSpec((1, C, C), lambda i, j: (i, 0, 0)),
        ],
        out_specs=[
            pl.BlockSpec((1, BN, C), lambda i, j: (i, j, 0)),
            pl.BlockSpec((1, BN, C), lambda i, j: (i, j, 0)),
        ],
        out_shape=[
            jax.ShapeDtypeStruct((NLAYERS, N, C), jnp.float32),
            jax.ShapeDtypeStruct((NLAYERS, N, C), jnp.float32),
        ],
    )(Z, f, Kf, K)


def _dense_out_body(pa_ref, pb_ref, k_ref, x_ref, z_ref):
    Ys = []
    for i in range(NLAYERS):
        agg = pa_ref[0, i] + pb_ref[0, i]
        y = -lax.dot_general(agg, k_ref[i], (((1,), (1,)), ((), ())),
                             preferred_element_type=jnp.float32)
        if i == NLAYERS - 1:
            y = y + x_ref[...]
        Ys.append(y)
    Yt = [None] * NLAYERS
    Yt[0] = math.sqrt(0.5) * Ys[0]
    for i in range(1, NLAYERS):
        a = math.sqrt((i + 1) / (i + 2))
        b = math.sqrt(i / (i + 1))
        Yt[i] = a * (b * Yt[i - 1] + Ys[i])
    W = [None] * NLAYERS
    W[NLAYERS - 1] = math.sqrt(NLAYERS / (NLAYERS + 1)) * Yt[NLAYERS - 1]
    for i in range(NLAYERS - 2, -1, -1):
        a = math.sqrt((i + 1) / (i + 2))
        W[i] = a * (a * W[i + 1] + Yt[i])
    for i in range(NLAYERS):
        z_ref[i] = W[i]


@jax.jit
def _dense_out(parts, K, X):
    return pl.pallas_call(
        _dense_out_body,
        grid=(N // BN,),
        in_specs=[
            pl.BlockSpec((1, NLAYERS, BN, C), lambda j: (0, 0, j, 0)),
            pl.BlockSpec((1, NLAYERS, BN, C), lambda j: (1, 0, j, 0)),
            pl.BlockSpec((NLAYERS, C, C), lambda j: (0, 0, 0)),
            pl.BlockSpec((BN, C), lambda j: (j, 0)),
        ],
        out_specs=pl.BlockSpec((NLAYERS, BN, C), lambda j: (0, j, 0)),
        out_shape=jax.ShapeDtypeStruct((NLAYERS, N, C), jnp.float32),
    )(parts, parts, K, X)


def kernel(X, f, edge_weight, K, K_features, edge_index):
    src = edge_index[0]
    dst = edge_index[1]
    zrows = jnp.zeros((RPS, C), jnp.float32)
    ewn = jnp.repeat(-edge_weight[:, None], L, axis=1).reshape(E * L)
    srcr = src.reshape(E // CH, CH)
    dstr = dst.reshape(E // CH, CH)
    idx3 = jnp.stack([
        jnp.stack([srcr, srcr + i * N, dstr + i * N], axis=1)
        for i in range(NLAYERS)
    ])
    Z = jnp.zeros((NLAYERS, N, C), dtype=jnp.float32)
    for _ in range(NFIX):
        dZ, nDZ = _dense_in(Z, f, K_features, K)
        parts = _edge_op(dZ.reshape(NLAYERS * N, C),
                         nDZ.reshape(NLAYERS * N, C), idx3, ewn, zrows)
        Z = _dense_out(parts, K, X)
    return (Z[-1], Z)
